# baseline (speedup 1.0000x reference)
import functools

import jax
import jax.numpy as jnp
from jax import lax
from jax.experimental import pallas as pl
from jax.experimental.pallas import tpu as pltpu
from jax.experimental.pallas import tpu_sc as plsc

N_NODE = 50000
N_EDGE = 800000
EMB = 100
PAD = 128
NCH = 8
CW = 16
NTILE = 16
WIN = 2048
NWIN = 25
EPT = WIN * NWIN
E_PAD = EPT * NTILE
NP = 50048
RPT = NP // NTILE


def _mm_body(x_ref, w_ref, o_ref):
    o_ref[...] = jnp.dot(x_ref[...].astype(jnp.bfloat16),
                         w_ref[...].astype(jnp.bfloat16),
                         preferred_element_type=jnp.float32)


def _norm_mm_body(h_ref, w_ref, n_ref, o_ref):
    h = h_ref[...]
    nrm = jnp.sqrt(jnp.sum(h * h, axis=-1, keepdims=True))
    n = h / jnp.maximum(nrm, 1e-12)
    n_ref[...] = n
    o_ref[...] = jnp.dot(h.astype(jnp.bfloat16),
                         w_ref[...].astype(jnp.bfloat16),
                         preferred_element_type=jnp.float32)


def _combine_body(x0_ref, n1_ref, h2_ref, o_ref):
    h2 = h2_ref[...]
    nrm = jnp.sqrt(jnp.sum(h2 * h2, axis=-1, keepdims=True))
    n2 = h2 / jnp.maximum(nrm, 1e-12)
    o_ref[...] = (x0_ref[...] + n1_ref[...] + n2) * (1.0 / 3.0)


_ROWS = 2000


def _tc_matmul(x, wp):
    return pl.pallas_call(
        _mm_body,
        grid=(N_NODE // _ROWS,),
        in_specs=[pl.BlockSpec((_ROWS, PAD), lambda i: (i, 0)),
                  pl.BlockSpec((PAD, PAD), lambda i: (0, 0))],
        out_specs=pl.BlockSpec((_ROWS, PAD), lambda i: (i, 0)),
        out_shape=jax.ShapeDtypeStruct((N_NODE, PAD), jnp.float32),
    )(x, wp)


def _tc_norm_matmul(h, wp):
    return pl.pallas_call(
        _norm_mm_body,
        grid=(N_NODE // _ROWS,),
        in_specs=[pl.BlockSpec((_ROWS, PAD), lambda i: (i, 0)),
                  pl.BlockSpec((PAD, PAD), lambda i: (0, 0))],
        out_specs=[pl.BlockSpec((_ROWS, PAD), lambda i: (i, 0)),
                   pl.BlockSpec((_ROWS, PAD), lambda i: (i, 0))],
        out_shape=[jax.ShapeDtypeStruct((N_NODE, PAD), jnp.float32),
                   jax.ShapeDtypeStruct((N_NODE, PAD), jnp.float32)],
    )(h, wp)


def _tc_combine(x0, n1, h2):
    return pl.pallas_call(
        _combine_body,
        grid=(N_NODE // _ROWS,),
        in_specs=[pl.BlockSpec((_ROWS, PAD), lambda i: (i, 0))] * 3,
        out_specs=pl.BlockSpec((_ROWS, PAD), lambda i: (i, 0)),
        out_shape=jax.ShapeDtypeStruct((N_NODE, PAD), jnp.float32),
    )(x0, n1, h2)


def _sc_body(xw16, cols8k, rows_p, w_p, zrows, h_cm,
             cbuf, rbuf, wbuf, gbuf, acc, sem_g, sem_s):
    c = lax.axis_index("c")
    s = lax.axis_index("s")
    t_row0 = s * (EPT // 128)

    for kk in range(NCH // 2):
        k = c * (NCH // 2) + kk
        cbase = k * (E_PAD // 128)

        pltpu.sync_copy(zrows, acc.at[pl.ds(s * RPT, RPT)])
        plsc.subcore_barrier()

        def fire_g(i, p):
            for j in range(WIN // 128):
                pltpu.async_copy(
                    xw16.at[cbuf.at[p * (WIN // 128) + j]],
                    gbuf.at[p, pl.ds(j * 128, 128)],
                    sem_g)

        def drain_g(i, p):
            for j in range(WIN // 128):
                pltpu.make_async_copy(
                    xw16.at[cbuf.at[p * (WIN // 128) + j]],
                    gbuf.at[p, pl.ds(j * 128, 128)],
                    sem_g).wait()

        def fire_s(i, p):
            for j in range(WIN // 128):
                pltpu.async_copy(
                    gbuf.at[p, pl.ds(j * 128, 128)],
                    acc.at[rbuf.at[p * (WIN // 128) + j]],
                    sem_s, add=True)

        def drain_s(i, p):
            for j in range(WIN // 128):
                pltpu.make_async_copy(
                    gbuf.at[p, pl.ds(j * 128, 128)],
                    acc.at[rbuf.at[p * (WIN // 128) + j]],
                    sem_s).wait()

        def stage_w(i, p):
            r0 = t_row0 + i * (WIN // 128)
            pltpu.sync_copy(cols8k.at[pl.ds(cbase + r0, WIN // 128)],
                            cbuf.at[pl.ds(p * (WIN // 128), WIN // 128)])
            pltpu.sync_copy(rows_p.at[pl.ds(r0, WIN // 128)],
                            rbuf.at[pl.ds(p * (WIN // 128), WIN // 128)])
            pltpu.sync_copy(w_p.at[pl.ds(r0, WIN // 128)],
                            wbuf.at[p])

        def mult(i, p):
            def jbody(j, _):
                for u0 in range(0, 128, 16):
                    wv = wbuf[p, j, pl.ds(u0, 16)]
                    for v in range(16):
                        r = j * 128 + u0 + v
                        gbuf[p, r, :] = gbuf[p, r, :] * wv[v]
                return 0
            lax.fori_loop(0, WIN // 128, jbody, 0)

        stage_w(0, 0)
        fire_g(0, 0)

        def wbody(i, _):
            p = lax.rem(i, 2)
            q = lax.rem(i + 1, 2)
            drain_g(i, p)

            @pl.when(i >= 1)
            def _():
                drain_s(i - 1, q)

            @pl.when(i + 1 < NWIN)
            def _():
                stage_w(i + 1, q)
                fire_g(i + 1, q)

            mult(i, p)
            fire_s(i, p)
            return 0

        lax.fori_loop(0, NWIN, wbody, 0)
        drain_s(NWIN - 1, (NWIN - 1) % 2)
        plsc.subcore_barrier()

        pltpu.sync_copy(acc.at[pl.ds(s * RPT, RPT)],
                        h_cm.at[pl.ds(k * NP + s * RPT, RPT)])
        plsc.subcore_barrier()


_sc_scatter = functools.partial(
    pl.kernel,
    mesh=plsc.VectorSubcoreMesh(core_axis_name="c", subcore_axis_name="s"),
    compiler_params=pltpu.CompilerParams(use_tc_tiling_on_sc=False),
    out_type=jax.ShapeDtypeStruct((NCH * NP, CW), jnp.float32),
    scratch_types=[
        pltpu.VMEM((2 * (WIN // 128), 128), jnp.int32),
        pltpu.VMEM((2 * (WIN // 128), 128), jnp.int32),
        pltpu.VMEM((2, WIN // 128, 128), jnp.float32),
        pltpu.VMEM((2, WIN, CW), jnp.float32),
        pltpu.VMEM_SHARED((NP, CW), jnp.float32),
        pltpu.SemaphoreType.DMA,
        pltpu.SemaphoreType.DMA,
    ],
)(_sc_body)


def _sc_segsum(xw, cols8k, rows_p, w_p, zrows):
    h_cm = _sc_scatter(xw.reshape(N_NODE * NCH, CW),
                       cols8k, rows_p, w_p, zrows)
    return jnp.moveaxis(h_cm.reshape(NCH, NP, CW)[:, :N_NODE], 0, 1).reshape(
        N_NODE, NCH * CW)


def kernel(embedding, edge_index, edge_weight, W0, W1):
    f32 = jnp.float32
    col = edge_index[1].astype(jnp.int32)
    row = edge_index[0].astype(jnp.int32)
    npad = E_PAD - N_EDGE
    spread = (jnp.arange(npad, dtype=jnp.int32) * 61) % N_NODE
    col_p = jnp.concatenate([col, spread])
    row_p = jnp.concatenate([row, spread])
    w_p = jnp.concatenate([edge_weight.astype(f32),
                           jnp.zeros((npad,), f32)])
    cols8k = (col_p[None, :] * NCH +
              jnp.arange(NCH, dtype=jnp.int32)[:, None]).reshape(-1, 128)
    rows_p2 = row_p.reshape(-1, 128)
    w_p2 = w_p.reshape(-1, 128)
    zrows = jnp.zeros((RPT, CW), f32)

    emb_p = jnp.pad(embedding, ((0, 0), (0, PAD - EMB)))
    w0p = jnp.pad(W0.T, ((0, PAD - EMB), (0, PAD - EMB)))
    w1p = jnp.pad(W1.T, ((0, PAD - EMB), (0, PAD - EMB)))

    xw0 = _tc_matmul(emb_p, w0p)
    h1 = _sc_segsum(xw0, cols8k, rows_p2, w_p2, zrows)
    n1, xw1 = _tc_norm_matmul(h1, w1p)
    h2 = _sc_segsum(xw1, cols8k, rows_p2, w_p2, zrows)
    out = _tc_combine(emb_p, n1, h2)
    return out[:, :EMB]


# speedup vs baseline: 1.6638x; 1.6638x over previous
import functools

import jax
import jax.numpy as jnp
from jax import lax
from jax.experimental import pallas as pl
from jax.experimental.pallas import tpu as pltpu
from jax.experimental.pallas import tpu_sc as plsc

N_NODE = 50000
N_EDGE = 800000
EMB = 100
PAD = 128
NCH = 8
CW = 16
NTILE = 16
WIN = 2048
NWIN = 25
EPT = WIN * NWIN
E_PAD = EPT * NTILE
NP = 50048
RPT = NP // NTILE


def _mm_body(x_ref, w_ref, o_ref):
    o_ref[...] = jnp.dot(x_ref[...].astype(jnp.bfloat16),
                         w_ref[...].astype(jnp.bfloat16),
                         preferred_element_type=jnp.float32)


def _norm_mm_body(h_ref, w_ref, n_ref, o_ref):
    h = h_ref[...]
    nrm = jnp.sqrt(jnp.sum(h * h, axis=-1, keepdims=True))
    n = h / jnp.maximum(nrm, 1e-12)
    n_ref[...] = n
    o_ref[...] = jnp.dot(h.astype(jnp.bfloat16),
                         w_ref[...].astype(jnp.bfloat16),
                         preferred_element_type=jnp.float32)


def _combine_body(x0_ref, n1_ref, h2_ref, o_ref):
    h2 = h2_ref[...]
    nrm = jnp.sqrt(jnp.sum(h2 * h2, axis=-1, keepdims=True))
    n2 = h2 / jnp.maximum(nrm, 1e-12)
    o_ref[...] = (x0_ref[...] + n1_ref[...] + n2) * (1.0 / 3.0)


_ROWS = 2000


def _tc_matmul(x, wp):
    return pl.pallas_call(
        _mm_body,
        grid=(N_NODE // _ROWS,),
        in_specs=[pl.BlockSpec((_ROWS, PAD), lambda i: (i, 0)),
                  pl.BlockSpec((PAD, PAD), lambda i: (0, 0))],
        out_specs=pl.BlockSpec((_ROWS, PAD), lambda i: (i, 0)),
        out_shape=jax.ShapeDtypeStruct((N_NODE, PAD), jnp.float32),
    )(x, wp)


def _tc_norm_matmul(h, wp):
    return pl.pallas_call(
        _norm_mm_body,
        grid=(N_NODE // _ROWS,),
        in_specs=[pl.BlockSpec((_ROWS, PAD), lambda i: (i, 0)),
                  pl.BlockSpec((PAD, PAD), lambda i: (0, 0))],
        out_specs=[pl.BlockSpec((_ROWS, PAD), lambda i: (i, 0)),
                   pl.BlockSpec((_ROWS, PAD), lambda i: (i, 0))],
        out_shape=[jax.ShapeDtypeStruct((N_NODE, PAD), jnp.float32),
                   jax.ShapeDtypeStruct((N_NODE, PAD), jnp.float32)],
    )(h, wp)


def _tc_combine(x0, n1, h2):
    return pl.pallas_call(
        _combine_body,
        grid=(N_NODE // _ROWS,),
        in_specs=[pl.BlockSpec((_ROWS, PAD), lambda i: (i, 0))] * 3,
        out_specs=pl.BlockSpec((_ROWS, PAD), lambda i: (i, 0)),
        out_shape=jax.ShapeDtypeStruct((N_NODE, PAD), jnp.float32),
    )(x0, n1, h2)


def _sc_body(xw16, cols8k, rows_p, w_p, zrows, h_cm,
             cbuf, rbuf, wbuf, gbuf, acc, sem_g, sem_s):
    c = lax.axis_index("c")
    s = lax.axis_index("s")
    t_row0 = s * (EPT // 128)

    for kk in range(NCH // 2):
        k = c * (NCH // 2) + kk
        cbase = k * (E_PAD // 128)

        pltpu.sync_copy(zrows, acc.at[pl.ds(s * RPT, RPT)])
        plsc.subcore_barrier()

        def fire_g(i, p):
            for j in range(WIN // 128):
                pltpu.async_copy(
                    xw16.at[cbuf.at[p * (WIN // 128) + j]],
                    gbuf.at[p, pl.ds(j * 128, 128)],
                    sem_g)

        def drain_g(i, p):
            for j in range(WIN // 128):
                pltpu.make_async_copy(
                    xw16.at[cbuf.at[p * (WIN // 128) + j]],
                    gbuf.at[p, pl.ds(j * 128, 128)],
                    sem_g).wait()

        def fire_s(i, p):
            for j in range(WIN // 128):
                pltpu.async_copy(
                    gbuf.at[p, pl.ds(j * 128, 128)],
                    acc.at[rbuf.at[p * (WIN // 128) + j]],
                    sem_s, add=True)

        def drain_s(i, p):
            for j in range(WIN // 128):
                pltpu.make_async_copy(
                    gbuf.at[p, pl.ds(j * 128, 128)],
                    acc.at[rbuf.at[p * (WIN // 128) + j]],
                    sem_s).wait()

        def stage_w(i, p):
            r0 = t_row0 + i * (WIN // 128)
            pltpu.sync_copy(cols8k.at[pl.ds(cbase + r0, WIN // 128)],
                            cbuf.at[pl.ds(p * (WIN // 128), WIN // 128)])
            pltpu.sync_copy(rows_p.at[pl.ds(r0, WIN // 128)],
                            rbuf.at[pl.ds(p * (WIN // 128), WIN // 128)])
            pltpu.sync_copy(w_p.at[pl.ds(r0, WIN // 128)],
                            wbuf.at[p])

        def mult(i, p):
            def mbody(g2, w2):
                def jbody(j, _):
                    for u0 in range(0, 128, 16):
                        wv = w2[j, pl.ds(u0, 16)]
                        for v in range(16):
                            r = j * 128 + u0 + v
                            g2[r, :] = g2[r, :] * wv[v]
                    return 0
                lax.fori_loop(0, WIN // 128, jbody, 0)

            @pl.when(p == 0)
            def _():
                mbody(gbuf.at[0], wbuf.at[0])

            @pl.when(p == 1)
            def _():
                mbody(gbuf.at[1], wbuf.at[1])

        stage_w(0, 0)
        fire_g(0, 0)

        def wbody(i, _):
            p = lax.rem(i, 2)
            q = lax.rem(i + 1, 2)
            drain_g(i, p)

            @pl.when(i >= 1)
            def _():
                drain_s(i - 1, q)

            @pl.when(i + 1 < NWIN)
            def _():
                stage_w(i + 1, q)
                fire_g(i + 1, q)

            mult(i, p)
            fire_s(i, p)
            return 0

        lax.fori_loop(0, NWIN, wbody, 0)
        drain_s(NWIN - 1, (NWIN - 1) % 2)
        plsc.subcore_barrier()

        pltpu.sync_copy(acc.at[pl.ds(s * RPT, RPT)],
                        h_cm.at[pl.ds(k * NP + s * RPT, RPT)])
        plsc.subcore_barrier()


_sc_scatter = functools.partial(
    pl.kernel,
    mesh=plsc.VectorSubcoreMesh(core_axis_name="c", subcore_axis_name="s"),
    compiler_params=pltpu.CompilerParams(use_tc_tiling_on_sc=False),
    out_type=jax.ShapeDtypeStruct((NCH * NP, CW), jnp.float32),
    scratch_types=[
        pltpu.VMEM((2 * (WIN // 128), 128), jnp.int32),
        pltpu.VMEM((2 * (WIN // 128), 128), jnp.int32),
        pltpu.VMEM((2, WIN // 128, 128), jnp.float32),
        pltpu.VMEM((2, WIN, CW), jnp.float32),
        pltpu.VMEM_SHARED((NP, CW), jnp.float32),
        pltpu.SemaphoreType.DMA,
        pltpu.SemaphoreType.DMA,
    ],
)(_sc_body)


def _sc_segsum(xw, cols8k, rows_p, w_p, zrows):
    h_cm = _sc_scatter(xw.reshape(N_NODE * NCH, CW),
                       cols8k, rows_p, w_p, zrows)
    return jnp.moveaxis(h_cm.reshape(NCH, NP, CW)[:, :N_NODE], 0, 1).reshape(
        N_NODE, NCH * CW)


def kernel(embedding, edge_index, edge_weight, W0, W1):
    f32 = jnp.float32
    col = edge_index[1].astype(jnp.int32)
    row = edge_index[0].astype(jnp.int32)
    npad = E_PAD - N_EDGE
    spread = (jnp.arange(npad, dtype=jnp.int32) * 61) % N_NODE
    col_p = jnp.concatenate([col, spread])
    row_p = jnp.concatenate([row, spread])
    w_p = jnp.concatenate([edge_weight.astype(f32),
                           jnp.zeros((npad,), f32)])
    cols8k = (col_p[None, :] * NCH +
              jnp.arange(NCH, dtype=jnp.int32)[:, None]).reshape(-1, 128)
    rows_p2 = row_p.reshape(-1, 128)
    w_p2 = w_p.reshape(-1, 128)
    zrows = jnp.zeros((RPT, CW), f32)

    emb_p = jnp.pad(embedding, ((0, 0), (0, PAD - EMB)))
    w0p = jnp.pad(W0.T, ((0, PAD - EMB), (0, PAD - EMB)))
    w1p = jnp.pad(W1.T, ((0, PAD - EMB), (0, PAD - EMB)))

    xw0 = _tc_matmul(emb_p, w0p)
    h1 = _sc_segsum(xw0, cols8k, rows_p2, w_p2, zrows)
    n1, xw1 = _tc_norm_matmul(h1, w1p)
    h2 = _sc_segsum(xw1, cols8k, rows_p2, w_p2, zrows)
    out = _tc_combine(emb_p, n1, h2)
    return out[:, :EMB]


# speedup vs baseline: 1.9853x; 1.1933x over previous
import functools

import jax
import jax.numpy as jnp
from jax import lax
from jax.experimental import pallas as pl
from jax.experimental.pallas import tpu as pltpu
from jax.experimental.pallas import tpu_sc as plsc

N_NODE = 50000
N_EDGE = 800000
EMB = 100
PAD = 128
NCH = 8
CW = 16
NTILE = 16
WIN = 1024
NWIN = 50
EPT = WIN * NWIN
E_PAD = EPT * NTILE
NP = 50048
RPT = NP // NTILE


def _mm_body(x_ref, w_ref, o_ref):
    o_ref[...] = jnp.dot(x_ref[...].astype(jnp.bfloat16),
                         w_ref[...].astype(jnp.bfloat16),
                         preferred_element_type=jnp.float32)


def _norm_mm_body(h_ref, w_ref, n_ref, o_ref):
    h = h_ref[...]
    nrm = jnp.sqrt(jnp.sum(h * h, axis=-1, keepdims=True))
    n = h / jnp.maximum(nrm, 1e-12)
    n_ref[...] = n
    o_ref[...] = jnp.dot(h.astype(jnp.bfloat16),
                         w_ref[...].astype(jnp.bfloat16),
                         preferred_element_type=jnp.float32)


def _combine_body(x0_ref, n1_ref, h2_ref, o_ref):
    h2 = h2_ref[...]
    nrm = jnp.sqrt(jnp.sum(h2 * h2, axis=-1, keepdims=True))
    n2 = h2 / jnp.maximum(nrm, 1e-12)
    o_ref[...] = (x0_ref[...] + n1_ref[...] + n2) * (1.0 / 3.0)


_ROWS = 2000


def _tc_matmul(x, wp):
    return pl.pallas_call(
        _mm_body,
        grid=(N_NODE // _ROWS,),
        in_specs=[pl.BlockSpec((_ROWS, PAD), lambda i: (i, 0)),
                  pl.BlockSpec((PAD, PAD), lambda i: (0, 0))],
        out_specs=pl.BlockSpec((_ROWS, PAD), lambda i: (i, 0)),
        out_shape=jax.ShapeDtypeStruct((N_NODE, PAD), jnp.float32),
    )(x, wp)


def _tc_norm_matmul(h, wp):
    return pl.pallas_call(
        _norm_mm_body,
        grid=(N_NODE // _ROWS,),
        in_specs=[pl.BlockSpec((_ROWS, PAD), lambda i: (i, 0)),
                  pl.BlockSpec((PAD, PAD), lambda i: (0, 0))],
        out_specs=[pl.BlockSpec((_ROWS, PAD), lambda i: (i, 0)),
                   pl.BlockSpec((_ROWS, PAD), lambda i: (i, 0))],
        out_shape=[jax.ShapeDtypeStruct((N_NODE, PAD), jnp.float32),
                   jax.ShapeDtypeStruct((N_NODE, PAD), jnp.float32)],
    )(h, wp)


def _tc_combine(x0, n1, h2):
    return pl.pallas_call(
        _combine_body,
        grid=(N_NODE // _ROWS,),
        in_specs=[pl.BlockSpec((_ROWS, PAD), lambda i: (i, 0))] * 3,
        out_specs=pl.BlockSpec((_ROWS, PAD), lambda i: (i, 0)),
        out_shape=jax.ShapeDtypeStruct((N_NODE, PAD), jnp.float32),
    )(x0, n1, h2)


def _sc_body(xw16, cols8k, rows_p, w_p, zrows, h_cm,
             cbuf, rbuf, wbuf, gbuf, acc, sem_ld, sem_g, sem_s):
    c = lax.axis_index("c")
    s = lax.axis_index("s")
    t_row0 = s * (EPT // 128)

    for kk in range(NCH // 2):
        k = c * (NCH // 2) + kk
        cbase = k * (E_PAD // 128)

        pltpu.sync_copy(zrows, acc.at[pl.ds(s * RPT, RPT)])
        plsc.subcore_barrier()

        def fire_g(i, p, s):
            for j in range(WIN // 128):
                pltpu.async_copy(
                    xw16.at[cbuf.at[s * (WIN // 128) + j]],
                    gbuf.at[p, pl.ds(j * 128, 128)],
                    sem_g)

        def drain_g(i, p, s):
            for j in range(WIN // 128):
                pltpu.make_async_copy(
                    xw16.at[cbuf.at[s * (WIN // 128) + j]],
                    gbuf.at[p, pl.ds(j * 128, 128)],
                    sem_g).wait()

        def fire_s(i, p, s):
            for j in range(WIN // 128):
                pltpu.async_copy(
                    gbuf.at[p, pl.ds(j * 128, 128)],
                    acc.at[rbuf.at[s * (WIN // 128) + j]],
                    sem_s, add=True)

        def drain_s(i, p, s):
            for j in range(WIN // 128):
                pltpu.make_async_copy(
                    gbuf.at[p, pl.ds(j * 128, 128)],
                    acc.at[rbuf.at[s * (WIN // 128) + j]],
                    sem_s).wait()

        def _stage_descs(i, s):
            r0 = t_row0 + i * (WIN // 128)
            return (
                pltpu.make_async_copy(
                    cols8k.at[pl.ds(cbase + r0, WIN // 128)],
                    cbuf.at[pl.ds(s * (WIN // 128), WIN // 128)], sem_ld),
                pltpu.make_async_copy(
                    rows_p.at[pl.ds(r0, WIN // 128)],
                    rbuf.at[pl.ds(s * (WIN // 128), WIN // 128)], sem_ld),
                pltpu.make_async_copy(
                    w_p.at[pl.ds(r0, WIN // 128)], wbuf.at[s], sem_ld),
            )

        def stage_fire(i, s):
            for d in _stage_descs(i, s):
                d.start()

        def stage_drain(i, s):
            for d in _stage_descs(i, s):
                d.wait()

        def mult(i, p):
            s = lax.rem(i, 3)

            def mbody(g2, w2):
                def jbody(j, _):
                    for u0 in range(0, 128, 16):
                        wv = w2[j, pl.ds(u0, 16)]
                        for v in range(16):
                            r = j * 128 + u0 + v
                            g2[r, :] = g2[r, :] * wv[v]
                    return 0
                lax.fori_loop(0, WIN // 128, jbody, 0)

            for pp in range(2):
                for ss in range(3):
                    @pl.when((p == pp) & (s == ss))
                    def _(pp=pp, ss=ss):
                        mbody(gbuf.at[pp], wbuf.at[ss])

        stage_fire(0, 0)
        stage_fire(1, 1)
        stage_drain(0, 0)
        fire_g(0, 0, 0)

        def wbody(i, _):
            p = lax.rem(i, 2)
            q = lax.rem(i + 1, 2)
            s = lax.rem(i, 3)
            s1 = lax.rem(i + 1, 3)
            s2 = lax.rem(i + 2, 3)

            @pl.when(i >= 1)
            def _():
                drain_s(i - 1, q, lax.rem(i + 2, 3))

            @pl.when(i + 2 < NWIN)
            def _():
                stage_fire(i + 2, s2)

            drain_g(i, p, s)

            @pl.when(i + 1 < NWIN)
            def _():
                stage_drain(i + 1, s1)
                fire_g(i + 1, q, s1)

            mult(i, p)
            fire_s(i, p, s)
            return 0

        lax.fori_loop(0, NWIN, wbody, 0)
        drain_s(NWIN - 1, (NWIN - 1) % 2, (NWIN - 1) % 3)
        plsc.subcore_barrier()

        pltpu.sync_copy(acc.at[pl.ds(s * RPT, RPT)],
                        h_cm.at[pl.ds(k * NP + s * RPT, RPT)])
        plsc.subcore_barrier()


_sc_scatter = functools.partial(
    pl.kernel,
    mesh=plsc.VectorSubcoreMesh(core_axis_name="c", subcore_axis_name="s"),
    compiler_params=pltpu.CompilerParams(use_tc_tiling_on_sc=False),
    out_type=jax.ShapeDtypeStruct((NCH * NP, CW), jnp.float32),
    scratch_types=[
        pltpu.VMEM((3 * (WIN // 128), 128), jnp.int32),
        pltpu.VMEM((3 * (WIN // 128), 128), jnp.int32),
        pltpu.VMEM((3, WIN // 128, 128), jnp.float32),
        pltpu.VMEM((2, WIN, CW), jnp.float32),
        pltpu.VMEM_SHARED((NP, CW), jnp.float32),
        pltpu.SemaphoreType.DMA,
        pltpu.SemaphoreType.DMA,
        pltpu.SemaphoreType.DMA,
    ],
)(_sc_body)


def _sc_segsum(xw, cols8k, rows_p, w_p, zrows):
    h_cm = _sc_scatter(xw.reshape(N_NODE * NCH, CW),
                       cols8k, rows_p, w_p, zrows)
    return jnp.moveaxis(h_cm.reshape(NCH, NP, CW)[:, :N_NODE], 0, 1).reshape(
        N_NODE, NCH * CW)


def kernel(embedding, edge_index, edge_weight, W0, W1):
    f32 = jnp.float32
    col = edge_index[1].astype(jnp.int32)
    row = edge_index[0].astype(jnp.int32)
    npad = E_PAD - N_EDGE
    spread = (jnp.arange(npad, dtype=jnp.int32) * 61) % N_NODE
    col_p = jnp.concatenate([col, spread])
    row_p = jnp.concatenate([row, spread])
    w_p = jnp.concatenate([edge_weight.astype(f32),
                           jnp.zeros((npad,), f32)])
    cols8k = (col_p[None, :] * NCH +
              jnp.arange(NCH, dtype=jnp.int32)[:, None]).reshape(-1, 128)
    rows_p2 = row_p.reshape(-1, 128)
    w_p2 = w_p.reshape(-1, 128)
    zrows = jnp.zeros((RPT, CW), f32)

    emb_p = jnp.pad(embedding, ((0, 0), (0, PAD - EMB)))
    w0p = jnp.pad(W0.T, ((0, PAD - EMB), (0, PAD - EMB)))
    w1p = jnp.pad(W1.T, ((0, PAD - EMB), (0, PAD - EMB)))

    xw0 = _tc_matmul(emb_p, w0p)
    h1 = _sc_segsum(xw0, cols8k, rows_p2, w_p2, zrows)
    n1, xw1 = _tc_norm_matmul(h1, w1p)
    h2 = _sc_segsum(xw1, cols8k, rows_p2, w_p2, zrows)
    out = _tc_combine(emb_p, n1, h2)
    return out[:, :EMB]
